# Initial kernel scaffold; baseline (speedup 1.0000x reference)
import jax
import jax.numpy as jnp
from jax.experimental import pallas as pl


def kernel(x, edge_index, edge_attr, W1, b1, W2, b2, Wc1, bc1, Wc2, bc2, Wf1, bf1, Wf2, bf2, Wf3, bf3):
    raise NotImplementedError("write your pallas kernel here")


# speedup vs baseline: 19.3469x; 19.3469x over previous
import functools
import jax
import jax.numpy as jnp
from jax import lax
from jax.experimental import pallas as pl
from jax.experimental.pallas import tpu as pltpu
from jax.experimental.pallas import tpu_sc as plsc

N = 129
E = 8192
F_IN = 128
CS = 256
NT = 16
EPT = E // NT
NG = EPT // 16
NF1 = 11648
NEG = -3.4e38


def _splat(v, dtype=jnp.int32):
    return jnp.full((16,), v, dtype=dtype)


def _iota():
    return lax.iota(jnp.int32, 16)


def _rsqrt16(x):
    xi = lax.bitcast_convert_type(x, jnp.int32)
    yi = _splat(0x5F3759DF) - (xi >> 1)
    y = lax.bitcast_convert_type(yi, jnp.float32)
    for _ in range(3):
        y = y * (1.5 - 0.5 * x * y * y)
    return y


def _elu16(v):
    return jnp.where(v > 0, v, jnp.exp(jnp.minimum(v, 0.0)) - 1.0)


def _bf16r(v):
    u = lax.bitcast_convert_type(v, jnp.int32)
    u = (u + 0x7FFF + ((u >> 16) & 1)) & jnp.int32(-65536)
    return lax.bitcast_convert_type(u, jnp.float32)


def _sc_body(
    xt9, srcr, dstr, ewr, w1h, scalh, wc1h, wc2h, wf1h, bf1h, wf2h, bf2h,
    wf3h, bf3h,
    outh,
    xtv, w1l, srcv, dstv, ewv, coefv, accv, partv, disv, xwv, xw2v, dloc,
    h1cv, xw2l, h2cv, h2m, stg, xwp, rl, rnall, sortv, ytv, ptv, ztv, qtv,
    scv, wc1v, wc2v, qall, wf1v, bf1l, wf2v, bf2l, wf3v, bf3l, h32v, h6v,
    outv,
    shpart, shxw, shdis, shh1, shxw2, shh2, shrank, shq,
):
    tid = lax.axis_index("s")
    it16 = _iota()

    pltpu.sync_copy(srcr.at[tid], srcv)
    pltpu.sync_copy(dstr.at[tid], dstv)
    pltpu.sync_copy(ewr.at[tid], ewv)

    @pl.when(tid < 9)
    def _():
        pltpu.sync_copy(xt9.at[tid], xtv)
        pltpu.sync_copy(w1h, w1l)

    @pl.when(tid < 3)
    def _():
        pltpu.sync_copy(scalh, scv)
        pltpu.sync_copy(wc1h, wc1v)
        pltpu.sync_copy(wc2h, wc2v)

    @pl.when(tid == 0)
    def _():
        pltpu.sync_copy(wf1h, wf1v)
        pltpu.sync_copy(bf1h, bf1l)
        pltpu.sync_copy(wf2h, wf2v)
        pltpu.sync_copy(bf2h, bf2l)
        pltpu.sync_copy(wf3h, wf3v)
        pltpu.sync_copy(bf3h, bf3l)

    @pl.when(tid < 9)
    def _():
        def body(j, acc):
            wj = plsc.load_gather(w1l, [_splat(0) + j])
            xv = plsc.load_gather(xtv, [_splat(0) + j * 16 + it16])
            return acc + wj * xv

        acc = lax.fori_loop(0, F_IN, body, jnp.zeros((16,), jnp.float32))
        stg[pl.ds(0, 16)] = acc
        pltpu.sync_copy(stg, shxw.at[tid])

    def zero_acc(g, _):
        accv[pl.ds(g * 16, 16)] = jnp.zeros((16,), jnp.float32)
        return 0

    lax.fori_loop(0, 48, zero_acc, 0)

    def deg_body(g, _):
        d16 = dstv[pl.ds(g * 16, 16)]
        for c in range(3):
            ew16 = ewv[pl.ds(c * EPT + g * 16, 16)]
            plsc.addupdate_scatter(accv, [d16 + c * CS], ew16)
        return 0

    lax.fori_loop(0, NG, deg_body, 0)
    pltpu.sync_copy(accv, shpart.at[tid])
    plsc.subcore_barrier()

    @pl.when(tid < 3)
    def _():
        pltpu.sync_copy(shpart, partv)
        for g in range(9):
            deg = jnp.full((16,), 1.0, jnp.float32)
            for t in range(NT):
                deg = deg + partv[t, pl.ds(tid * CS + g * 16, 16)]
            dloc[pl.ds(g * 16, 16)] = _rsqrt16(deg)
        pltpu.sync_copy(dloc, shdis.at[pl.ds(tid * CS, CS)])
    plsc.subcore_barrier()

    pltpu.sync_copy(shdis, disv)
    pltpu.sync_copy(shxw, xwp)
    for g in range(9):
        xwv[pl.ds(g * 16, 16)] = xwp[g, pl.ds(0, 16)]
    lax.fori_loop(0, 48, zero_acc, 0)

    def msg1_body(g, _):
        s16 = srcv[pl.ds(g * 16, 16)]
        d16 = dstv[pl.ds(g * 16, 16)]
        xws = plsc.load_gather(xwv, [s16])
        for c in range(3):
            ew16 = ewv[pl.ds(c * EPT + g * 16, 16)]
            dis_s = plsc.load_gather(disv, [s16 + c * CS])
            dis_d = plsc.load_gather(disv, [d16 + c * CS])
            coef = dis_s * ew16 * dis_d
            coefv[pl.ds(c * EPT + g * 16, 16)] = coef
            plsc.addupdate_scatter(accv, [d16 + c * CS], coef * xws)
        return 0

    lax.fori_loop(0, NG, msg1_body, 0)
    pltpu.sync_copy(accv, shpart.at[tid])
    plsc.subcore_barrier()

    @pl.when(tid < 3)
    def _():
        pltpu.sync_copy(shpart, partv)
        b1s = scv[pl.ds(0, 16)]
        w2s = scv[pl.ds(32, 16)]
        for g in range(9):
            agg = jnp.zeros((16,), jnp.float32)
            for t in range(NT):
                agg = agg + partv[t, pl.ds(tid * CS + g * 16, 16)]
            dis = disv[pl.ds(tid * CS + g * 16, 16)]
            xw = xwv[pl.ds(g * 16, 16)]
            h1 = agg + dis * dis * xw + b1s
            h1cv[pl.ds(g * 16, 16)] = h1
            xw2l[pl.ds(g * 16, 16)] = h1 * w2s
        pltpu.sync_copy(h1cv, shh1.at[pl.ds(tid * CS, CS)])
        pltpu.sync_copy(xw2l, shxw2.at[pl.ds(tid * CS, CS)])
    plsc.subcore_barrier()

    pltpu.sync_copy(shxw2, xw2v)
    lax.fori_loop(0, 48, zero_acc, 0)

    def msg2_body(g, _):
        s16 = srcv[pl.ds(g * 16, 16)]
        d16 = dstv[pl.ds(g * 16, 16)]
        for c in range(3):
            coef = coefv[pl.ds(c * EPT + g * 16, 16)]
            m2 = coef * plsc.load_gather(xw2v, [s16 + c * CS])
            plsc.addupdate_scatter(accv, [d16 + c * CS], m2)
        return 0

    lax.fori_loop(0, NG, msg2_body, 0)
    pltpu.sync_copy(accv, shpart.at[tid])
    plsc.subcore_barrier()

    @pl.when(tid < 3)
    def _():
        pltpu.sync_copy(shpart, partv)
        b2s = scv[pl.ds(16, 16)]
        for g in range(9):
            agg = jnp.zeros((16,), jnp.float32)
            for t in range(NT):
                agg = agg + partv[t, pl.ds(tid * CS + g * 16, 16)]
            dis = disv[pl.ds(tid * CS + g * 16, 16)]
            xw2 = xw2v[pl.ds(tid * CS + g * 16, 16)]
            h2cv[pl.ds(g * 16, 16)] = agg + dis * dis * xw2 + b2s
        pltpu.sync_copy(h2cv, shh2.at[pl.ds(tid * CS, CS)])
    plsc.subcore_barrier()

    @pl.when(tid < 9)
    def _():
        pltpu.sync_copy(shh2, h2m)
        for c in range(3):
            v = h2m[pl.ds(c * CS + 128, 16)]
            h2m[pl.ds(c * CS + 128, 16)] = jnp.where(it16 < 1, v, NEG)
        ivec = tid * 16 + it16
        for c in range(3):
            si = h2m[pl.ds(c * CS + tid * 16, 16)]

            def rank_body(j, cnt, c=c, si=si, ivec=ivec):
                hj = plsc.load_gather(h2m, [_splat(c * CS) + j])
                gt = hj > si
                eq = (hj == si) & (_splat(0) + j < ivec)
                return (cnt + jnp.where(gt, 1, 0).astype(jnp.int32)
                        + jnp.where(eq, 1, 0).astype(jnp.int32))

            cnt = lax.fori_loop(0, N, rank_body, jnp.zeros((16,), jnp.int32))
            rl[pl.ds(c * 16, 16)] = cnt
        pltpu.sync_copy(rl, shrank.at[tid])
    plsc.subcore_barrier()

    @pl.when(tid < 3)
    def _():
        pltpu.sync_copy(shrank, rnall)
        pltpu.sync_copy(shh1.at[pl.ds(tid * CS, CS)], h1cv)
        pltpu.sync_copy(shh2.at[pl.ds(tid * CS, CS)], h2cv)
        for g in range(18):
            sortv[pl.ds(g * 16, 16)] = jnp.zeros((16,), jnp.float32)
        for g in range(9):
            r16 = rnall[g, pl.ds(tid * 16, 16)]
            valid = (g * 16 + it16) < N
            plsc.store_scatter(sortv, [r16], _bf16r(h1cv[pl.ds(g * 16, 16)]),
                               mask=valid)
            plsc.store_scatter(sortv, [r16 + 144],
                               _bf16r(h2cv[pl.ds(g * 16, 16)]), mask=valid)

        def conv1_body(tg, _):
            idxb = _splat(0) + tg * 16 + it16
            for o in range(3):
                acc = scv[pl.ds(64 + o * 16, 16)]
                for i in range(2):
                    for k in range(3):
                        w = wc1v[pl.ds((o * 6 + i * 3 + k) * 16, 16)]
                        hv = plsc.load_gather(sortv, [idxb + (i * 144 + k)])
                        acc = acc + w * hv
                ytv[pl.ds(o * 144 + tg * 16, 16)] = acc
            return 0

        lax.fori_loop(0, 8, conv1_body, 0)

        def pool1_body(tg, _):
            idxb = _splat(0) + tg * 16 + it16
            for o in range(3):
                m = jnp.maximum(
                    jnp.maximum(
                        plsc.load_gather(ytv, [idxb + o * 144]),
                        plsc.load_gather(ytv, [idxb + (o * 144 + 1)])),
                    plsc.load_gather(ytv, [idxb + (o * 144 + 2)]))
                ptv[pl.ds(o * 144 + tg * 16, 16)] = _bf16r(m)
            return 0

        lax.fori_loop(0, 8, pool1_body, 0)

        def conv2_body(tg, _):
            idxb = _splat(0) + tg * 16 + it16
            acc = scv[pl.ds(48, 16)]
            for i in range(3):
                for k in range(3):
                    w = wc2v[pl.ds((i * 3 + k) * 16, 16)]
                    acc = acc + w * plsc.load_gather(ptv, [idxb + (i * 144 + k)])
            ztv[pl.ds(tg * 16, 16)] = acc
            return 0

        lax.fori_loop(0, 8, conv2_body, 0)

        def pool2_body(tg, _):
            idxb = _splat(0) + tg * 16 + it16
            q = jnp.maximum(
                jnp.maximum(plsc.load_gather(ztv, [idxb]),
                            plsc.load_gather(ztv, [idxb + 1])),
                plsc.load_gather(ztv, [idxb + 2]))
            qtv[pl.ds(tg * 16, 16)] = _bf16r(q)
            return 0

        lax.fori_loop(0, 8, pool2_body, 0)
        pltpu.sync_copy(qtv, shq.at[pl.ds(tid * CS, CS)])
    plsc.subcore_barrier()

    @pl.when(tid == 0)
    def _():
        pltpu.sync_copy(shq, qall)
        a0 = bf1l[pl.ds(0, 16)]
        a1 = bf1l[pl.ds(16, 16)]

        def mlp1_body(t, carry, c=0):
            a0, a1 = carry
            xv = plsc.load_gather(qall, [_splat(c * CS) + t])
            bidx = _splat(c * 121 * 32) + t * 32 + it16
            w0 = plsc.load_gather(wf1v, [bidx])
            w1 = plsc.load_gather(wf1v, [bidx + 16])
            return (a0 + xv * w0, a1 + xv * w1)

        for c in range(3):
            a0, a1 = lax.fori_loop(0, 121,
                                   functools.partial(mlp1_body, c=c), (a0, a1))
        h32v[pl.ds(0, 16)] = _bf16r(_elu16(a0))
        h32v[pl.ds(16, 16)] = _bf16r(_elu16(a1))

        def mlp2_body(j, acc):
            xj = plsc.load_gather(h32v, [_splat(0) + j])
            wr = plsc.load_gather(wf2v, [_splat(0) + j * 16 + it16])
            return acc + xj * wr

        h6 = _bf16r(_elu16(lax.fori_loop(0, 32, mlp2_body, bf2l[...])))
        h6v[...] = h6

        def mlp3_body(j, acc):
            xj = plsc.load_gather(h6v, [_splat(0) + j])
            wr = plsc.load_gather(wf3v, [_splat(0) + j * 16 + it16])
            return acc + xj * wr

        outv[...] = lax.fori_loop(0, 6, mlp3_body, bf3l[...])
        pltpu.sync_copy(outv, outh)


@jax.jit
def _dgcnn_sc(xt9, srcr, dstr, ewr, w1h, scalh, wc1h, wc2h, wf1h, bf1h,
              wf2h, bf2h, wf3h, bf3h):
    f32 = jnp.float32
    i32 = jnp.int32
    mesh = plsc.VectorSubcoreMesh(core_axis_name="c", subcore_axis_name="s",
                                  num_cores=1)
    scratch = [
        pltpu.VMEM((F_IN * 16,), f32),
        pltpu.VMEM((F_IN,), f32),
        pltpu.VMEM((EPT,), i32),
        pltpu.VMEM((EPT,), i32),
        pltpu.VMEM((3 * EPT,), f32),
        pltpu.VMEM((3 * EPT,), f32),
        pltpu.VMEM((3 * CS,), f32),
        pltpu.VMEM((NT, 3 * CS), f32),
        pltpu.VMEM((3 * CS,), f32),
        pltpu.VMEM((CS,), f32),
        pltpu.VMEM((3 * CS,), f32),
        pltpu.VMEM((CS,), f32),
        pltpu.VMEM((CS,), f32),
        pltpu.VMEM((CS,), f32),
        pltpu.VMEM((CS,), f32),
        pltpu.VMEM((3 * CS,), f32),
        pltpu.VMEM((128,), f32),
        pltpu.VMEM((NT, 128), f32),
        pltpu.VMEM((128,), i32),
        pltpu.VMEM((NT, 128), i32),
        pltpu.VMEM((2 * 144,), f32),
        pltpu.VMEM((3 * 144,), f32),
        pltpu.VMEM((3 * 144,), f32),
        pltpu.VMEM((144,), f32),
        pltpu.VMEM((CS,), f32),
        pltpu.VMEM((128,), f32),
        pltpu.VMEM((384,), f32),
        pltpu.VMEM((256,), f32),
        pltpu.VMEM((3 * CS,), f32),
        pltpu.VMEM((NF1,), f32),
        pltpu.VMEM((32,), f32),
        pltpu.VMEM((32 * 16,), f32),
        pltpu.VMEM((16,), f32),
        pltpu.VMEM((6 * 16,), f32),
        pltpu.VMEM((16,), f32),
        pltpu.VMEM((32,), f32),
        pltpu.VMEM((16,), f32),
        pltpu.VMEM((16,), f32),
        pltpu.VMEM_SHARED((NT, 3 * CS), f32),
        pltpu.VMEM_SHARED((NT, 128), f32),
        pltpu.VMEM_SHARED((3 * CS,), f32),
        pltpu.VMEM_SHARED((3 * CS,), f32),
        pltpu.VMEM_SHARED((3 * CS,), f32),
        pltpu.VMEM_SHARED((3 * CS,), f32),
        pltpu.VMEM_SHARED((NT, 128), i32),
        pltpu.VMEM_SHARED((3 * CS,), f32),
    ]
    run = pl.kernel(
        _sc_body,
        out_type=jax.ShapeDtypeStruct((16,), f32),
        mesh=mesh,
        scratch_types=scratch,
        compiler_params=pltpu.CompilerParams(needs_layout_passes=False),
    )
    return run(xt9, srcr, dstr, ewr, w1h, scalh, wc1h, wc2h, wf1h, bf1h,
               wf2h, bf2h, wf3h, bf3h)


def _bfr_host(a):
    u = lax.bitcast_convert_type(a.astype(jnp.float32), jnp.int32)
    u = (u + 0x7FFF + ((u >> 16) & 1)) & jnp.int32(-65536)
    return lax.bitcast_convert_type(u, jnp.float32)


def kernel(x, edge_index, edge_attr, W1, b1, W2, b2, Wc1, bc1, Wc2, bc2,
           Wf1, bf1, Wf2, bf2, Wf3, bf3):
    f32 = jnp.float32
    xbf = _bfr_host(x)
    xp = jnp.zeros((144, F_IN), f32).at[:N].set(xbf)
    xt9 = xp.T.reshape(F_IN, 9, 16).transpose(1, 0, 2).reshape(9, F_IN * 16)
    srcr = edge_index[0].astype(jnp.int32).reshape(NT, EPT)
    dstr = edge_index[1].astype(jnp.int32).reshape(NT, EPT)
    ewr = (edge_attr.astype(f32).T.reshape(3, NT, EPT)
           .transpose(1, 0, 2).reshape(NT, 3 * EPT))
    w1h = _bfr_host(W1).reshape(F_IN)
    svals = jnp.concatenate([
        b1.astype(f32)[:1], b2.astype(f32)[:1],
        W2.astype(f32).reshape(1), bc2.astype(f32)[:1],
        bc1.astype(f32), jnp.zeros((1,), f32)])
    scalh = jnp.broadcast_to(svals[:, None], (8, 16)).reshape(128)
    bfr = _bfr_host
    wc1p = jnp.zeros((24,), f32).at[:18].set(bfr(Wc1).reshape(18))
    wc1h = jnp.broadcast_to(wc1p[:, None], (24, 16)).reshape(384)
    wc2p = jnp.zeros((16,), f32).at[:9].set(bfr(Wc2).reshape(9))
    wc2h = jnp.broadcast_to(wc2p[:, None], (16, 16)).reshape(256)
    wf1h = jnp.zeros((NF1,), f32).at[:363 * 32].set(
        bfr(Wf1).reshape(363 * 32))
    bf1h = bf1.astype(f32)
    wf2h = (jnp.zeros((32, 16), f32).at[:, :6].set(bfr(Wf2))
            .reshape(32 * 16))
    bf2h = jnp.zeros((16,), f32).at[:6].set(bf2.astype(f32))
    wf3h = jnp.zeros((6, 16), f32).at[:, :2].set(bfr(Wf3)).reshape(96)
    bf3h = jnp.zeros((16,), f32).at[:2].set(bf3.astype(f32))
    out = _dgcnn_sc(xt9, srcr, dstr, ewr, w1h, scalh, wc1h, wc2h, wf1h,
                    bf1h, wf2h, bf2h, wf3h, bf3h)
    return out[:2].reshape(1, 2)
